# baseline (speedup 1.0000x reference)
import numpy as np
import jax
import jax.numpy as jnp
from jax import lax
from jax.experimental import pallas as pl
from jax.experimental.pallas import tpu as pltpu
from jax.experimental.pallas import tpu_sc as plsc

_N = 8388608
_NC, _NS = 2, 16
_C = 16384
_NSTEP = _N // _C
_S1 = _NSTEP // _NS
_T = 16384
_D = _N // _T
_HB = _D // 2
_BPW = _HB // _NS
_L = 16

_U32 = np.uint32


def _threefry2x32(k1, k2, x1, x2):
    rot = ((13, 15, 26, 6), (17, 29, 16, 24))
    ks = (k1, k2, _U32(k1 ^ k2 ^ _U32(0x1BD11BDA)))
    x = [(x1 + ks[0]).astype(_U32), (x2 + ks[1]).astype(_U32)]
    for i in range(1, 6):
        for d in rot[(i - 1) % 2]:
            x[0] = (x[0] + x[1]).astype(_U32)
            x[1] = ((x[1] << _U32(d)) | (x[1] >> _U32(32 - d))).astype(_U32)
            x[1] = x[0] ^ x[1]
        x[0] = (x[0] + ks[i % 3]).astype(_U32)
        x[1] = (x[1] + ks[(i + 1) % 3] + _U32(i)).astype(_U32)
    return x


def _fixed_perm(seed, n):
    key = (_U32(0), _U32(seed))
    x = np.arange(n, dtype=np.int64)
    num_rounds = int(np.ceil(3 * np.log(n) / np.log(np.iinfo(np.uint32).max)))
    for _ in range(num_rounds):
        hi, lo = np.zeros(2, _U32), np.arange(2, dtype=_U32)
        b1, b2 = _threefry2x32(key[0], key[1], hi, lo)
        key, subkey = (b1[0], b2[0]), (b1[1], b2[1])
        chi = np.zeros(n, _U32)
        clo = np.arange(n, dtype=np.uint64).astype(_U32)
        s1, s2 = _threefry2x32(subkey[0], subkey[1], chi, clo)
        x = x[np.argsort(s1 ^ s2, kind="stable")]
    return x


_sched_cache = []


def _schedule():
    if _sched_cache:
        return _sched_cache[0]
    perm = _fixed_perm(42, _N)
    inv = np.empty(_N, np.int64)
    inv[perm] = np.arange(_N)
    bj = inv // _T

    keeps = {}
    maxcnt = 0
    for g in range(_NSTEP):
        sl = slice(g * _C, (g + 1) * _C)
        bjg = bj[sl]
        order = np.argsort(bjg, kind="stable").astype(np.int32)
        cut = int(np.searchsorted(bjg[order], _HB))
        keeps[(0, g)] = order[:cut]
        keeps[(1, g)] = order[cut:]
        maxcnt = max(maxcnt, cut, _C - cut)
    kp = -(-maxcnt // 128) * 128
    kl = np.zeros((2, _NSTEP, kp), np.int32)
    zpos = np.empty(_N, np.int64)
    for c in range(2):
        for g in range(_NSTEP):
            k = keeps[(c, g)]
            kl[c, g, : len(k)] = k
            zpos[g * _C + k] = (c * _NSTEP + g) * kp + np.arange(len(k))

    p = zpos[perm]
    prow = p // _L
    plane = (p % _L).astype(np.int32)
    rows_list = []
    l2 = np.empty(_N, np.int32)
    for b in range(_D):
        sl = slice(b * _T, (b + 1) * _T)
        rows = np.unique(prow[sl])
        rows_list.append(rows)
        l2[sl] = np.searchsorted(rows, prow[sl]).astype(np.int32) * _L + plane[sl]
    rmax = max(len(r) for r in rows_list)
    r_pad = -(-rmax // 8) * 8
    rl = np.zeros((_D, r_pad), np.int32)
    for b, rows in enumerate(rows_list):
        rl[b, : len(rows)] = rows
        c = b // _HB
        assert rows.min() >= c * _NSTEP * kp // _L
        assert rows.max() < (c + 1) * _NSTEP * kp // _L
    _sched_cache.append(
        (kl.reshape(-1), rl.reshape(-1), l2, kp, r_pad))
    return _sched_cache[0]


def kernel(x):
    kl_np, rl_np, l2_np, kp, r_pad = _schedule()
    kprow = kp // _L
    zrows = 2 * _NSTEP * kprow
    mesh = plsc.VectorSubcoreMesh(core_axis_name="c", subcore_axis_name="s")
    cparams = pltpu.CompilerParams(
        needs_layout_passes=False, use_tc_tiling_on_sc=False)

    def _body(x_hbm, kl_hbm, rl_hbm, l2_hbm, out_hbm, z_hbm,
              s0, s1, s2, s3, s4, s5, s6, s7):
        c = lax.axis_index("c")
        t = lax.axis_index("s")
        semA = (s0, s1)
        semB = (s2, s3)
        semC = (s4, s5)
        semO = (s6, s7)

        def pass1(src0, src1, kv0, kv1, st0, st1):
            src = (src0, src1)
            kv = (kv0, kv1)
            st = (st0, st1)

            def in_copies(i):
                g = t * _S1 + i
                return (
                    pltpu.async_copy(
                        x_hbm.at[pl.ds(g * _C, _C)], src[i % 2], semA[i % 2]),
                    pltpu.async_copy(
                        kl_hbm.at[pl.ds((c * _NSTEP + g) * kp, kp)],
                        kv[i % 2], semB[i % 2]),
                )

            ics = {0: in_copies(0)}
            oc = {}
            for i in range(_S1):
                if i + 1 < _S1:
                    ics[i + 1] = in_copies(i + 1)
                for d in ics.pop(i):
                    d.wait()
                if i >= 2:
                    oc[i - 2].wait()
                cur = i % 2
                src_r, kv_r, st_r = src[cur], kv[cur], st[cur]

                @plsc.parallel_loop(0, kprow, unroll=8)
                def _(k, src_r=src_r, kv_r=kv_r, st_r=st_r):
                    idx16 = kv_r[pl.ds(k * _L, _L)]
                    st_r[k] = plsc.load_gather(src_r, [idx16])

                g = t * _S1 + i
                oc[i] = pltpu.async_copy(
                    st_r, z_hbm.at[pl.ds((c * _NSTEP + g) * kprow, kprow)],
                    semO[cur])
            oc[_S1 - 2].wait()
            oc[_S1 - 1].wait()

        pl.run_scoped(
            pass1,
            pltpu.VMEM((_C,), jnp.float32),
            pltpu.VMEM((_C,), jnp.float32),
            pltpu.VMEM((kp,), jnp.int32),
            pltpu.VMEM((kp,), jnp.int32),
            pltpu.VMEM((kprow, _L), jnp.float32),
            pltpu.VMEM((kprow, _L), jnp.float32),
        )

        plsc.subcore_barrier()

        def pass2(rl0, rl1, rw0, rw1, li0, li1, ob0, ob1):
            rlb = (rl0, rl1)
            rw = (rw0, rw1)
            li = (li0, li1)
            ob = (ob0, ob1)

            def rl_copy(s):
                blk = c * _HB + t * _BPW + s
                return pltpu.async_copy(
                    rl_hbm.at[pl.ds(blk * r_pad, r_pad)],
                    rlb[s % 2], semA[s % 2])

            def l2_copy(s):
                blk = c * _HB + t * _BPW + s
                return pltpu.async_copy(
                    l2_hbm.at[pl.ds(blk * _T, _T)], li[s % 2], semB[s % 2])

            def row_gather(s):
                return pltpu.async_copy(
                    z_hbm.at[rlb[s % 2]], rw[s % 2], semC[s % 2])

            def out_copy(s):
                blk = c * _HB + t * _BPW + s
                return pltpu.async_copy(
                    ob[s % 2], out_hbm.at[pl.ds(blk * _T, _T)], semO[s % 2])

            rlc = {0: rl_copy(0)}
            l2c = {0: l2_copy(0)}
            rlc[0].wait()
            rg = {0: row_gather(0)}
            rlc[1] = rl_copy(1)
            l2c[1] = l2_copy(1)
            oc = {}
            for s in range(_BPW):
                if s + 1 < _BPW:
                    rlc[s + 1].wait()
                    rg[s + 1] = row_gather(s + 1)
                rg[s].wait()
                l2c[s].wait()
                if s >= 2:
                    oc[s - 2].wait()
                cur = s % 2
                rw_r, li_r, ob_r = rw[cur], li[cur], ob[cur]

                @plsc.parallel_loop(0, _T // _L, unroll=8)
                def _(k, rw_r=rw_r, li_r=li_r, ob_r=ob_r):
                    idx16 = li_r[pl.ds(k * _L, _L)]
                    r16 = lax.shift_right_logical(idx16, 4)
                    c16 = lax.bitwise_and(idx16, 15)
                    ob_r[pl.ds(k * _L, _L)] = plsc.load_gather(
                        rw_r, [r16, c16])

                oc[s] = out_copy(s)
                if s + 2 < _BPW:
                    rlc[s + 2] = rl_copy(s + 2)
                    l2c[s + 2] = l2_copy(s + 2)
            oc[_BPW - 2].wait()
            oc[_BPW - 1].wait()

        pl.run_scoped(
            pass2,
            pltpu.VMEM((r_pad,), jnp.int32),
            pltpu.VMEM((r_pad,), jnp.int32),
            pltpu.VMEM((r_pad, _L), jnp.float32),
            pltpu.VMEM((r_pad, _L), jnp.float32),
            pltpu.VMEM((_T,), jnp.int32),
            pltpu.VMEM((_T,), jnp.int32),
            pltpu.VMEM((_T,), jnp.float32),
            pltpu.VMEM((_T,), jnp.float32),
        )

    f = pl.kernel(
        _body,
        out_type=(
            jax.ShapeDtypeStruct((_N,), jnp.float32),
            jax.ShapeDtypeStruct((zrows, _L), jnp.float32),
        ),
        mesh=mesh,
        compiler_params=cparams,
        scratch_types=[pltpu.SemaphoreType.DMA] * 8,
    )
    out, _ = f(x, jnp.asarray(kl_np), jnp.asarray(rl_np), jnp.asarray(l2_np))
    return out


# speedup vs baseline: 2.0919x; 2.0919x over previous
import numpy as np
import jax
import jax.numpy as jnp
from jax import lax
from jax.experimental import pallas as pl
from jax.experimental.pallas import tpu as pltpu
from jax.experimental.pallas import tpu_sc as plsc

_N = 8388608
_NC, _NS = 2, 16
_NW = _NC * _NS
_PER_W = _N // _NW
_CHUNK = 16384
_STEPS = _PER_W // _CHUNK

_U32 = np.uint32


def _threefry2x32(k1, k2, x1, x2):
    rot = ((13, 15, 26, 6), (17, 29, 16, 24))
    ks = (k1, k2, _U32(k1 ^ k2 ^ _U32(0x1BD11BDA)))
    x = [(x1 + ks[0]).astype(_U32), (x2 + ks[1]).astype(_U32)]
    for i in range(1, 6):
        for d in rot[(i - 1) % 2]:
            x[0] = (x[0] + x[1]).astype(_U32)
            x[1] = ((x[1] << _U32(d)) | (x[1] >> _U32(32 - d))).astype(_U32)
            x[1] = x[0] ^ x[1]
        x[0] = (x[0] + ks[i % 3]).astype(_U32)
        x[1] = (x[1] + ks[(i + 1) % 3] + _U32(i)).astype(_U32)
    return x


def _fixed_perm(seed, n):
    key = (_U32(0), _U32(seed))
    x = np.arange(n, dtype=np.int32)
    num_rounds = int(np.ceil(3 * np.log(n) / np.log(np.iinfo(np.uint32).max)))
    for _ in range(num_rounds):
        hi, lo = np.zeros(2, _U32), np.arange(2, dtype=_U32)
        b1, b2 = _threefry2x32(key[0], key[1], hi, lo)
        key, subkey = (b1[0], b2[0]), (b1[1], b2[1])
        chi = np.zeros(n, _U32)
        clo = np.arange(n, dtype=np.uint64).astype(_U32)
        s1, s2 = _threefry2x32(subkey[0], subkey[1], chi, clo)
        x = x[np.argsort(s1 ^ s2, kind="stable")]
    return x


_perm_const = []


def _perm_i32():
    if not _perm_const:
        _perm_const.append(_fixed_perm(42, _N))
    return _perm_const[0]


def _gather_body(perm_hbm, x_hbm, out_hbm, idx0, idx1, idx2,
                 dat0, dat1, dat2, isem, gsem0, gsem1, osem0, osem1):
    wid = lax.axis_index("s") * _NC + lax.axis_index("c")
    base = wid * _PER_W
    idx = (idx0, idx1, idx2)
    dat = (dat0, dat1, dat2)
    gsem = (gsem0, gsem1)
    osem = (osem0, osem1)

    def idx_copy(s):
        src = perm_hbm.at[pl.ds(base + s * _CHUNK, _CHUNK)]
        return pltpu.async_copy(src, idx[s % 3], isem)

    def gather(s):
        return pltpu.async_copy(x_hbm.at[idx[s % 3]], dat[s % 3], gsem[s % 2])

    def out_copy(s):
        dst = out_hbm.at[pl.ds(base + s * _CHUNK, _CHUNK)]
        return pltpu.async_copy(dat[s % 3], dst, osem[s % 2])

    ic0 = idx_copy(0)
    ic0.wait()
    g = {0: gather(0)}
    ic1 = idx_copy(1)
    ic1.wait()
    g[1] = gather(1)
    oc = {}
    for s in range(_STEPS):
        g[s].wait()
        oc[s] = out_copy(s)
        if s + 2 < _STEPS:
            ic = idx_copy(s + 2)
            ic.wait()
            if s >= 1:
                oc[s - 1].wait()
            g[s + 2] = gather(s + 2)
    oc[_STEPS - 3].wait()
    oc[_STEPS - 2].wait()
    oc[_STEPS - 1].wait()


def kernel(x):
    perm = jnp.asarray(_perm_i32())
    mesh = plsc.VectorSubcoreMesh(core_axis_name="c", subcore_axis_name="s")
    f = pl.kernel(
        _gather_body,
        out_type=jax.ShapeDtypeStruct((_N,), jnp.float32),
        mesh=mesh,
        scratch_types=[
            pltpu.VMEM((_CHUNK,), jnp.int32),
            pltpu.VMEM((_CHUNK,), jnp.int32),
            pltpu.VMEM((_CHUNK,), jnp.int32),
            pltpu.VMEM((_CHUNK,), jnp.float32),
            pltpu.VMEM((_CHUNK,), jnp.float32),
            pltpu.VMEM((_CHUNK,), jnp.float32),
            pltpu.SemaphoreType.DMA,
            pltpu.SemaphoreType.DMA,
            pltpu.SemaphoreType.DMA,
            pltpu.SemaphoreType.DMA,
            pltpu.SemaphoreType.DMA,
        ],
    )
    return f(perm, x)
